# baseline (speedup 1.0000x reference)
import functools

import jax
import jax.numpy as jnp
from jax import lax
from jax.experimental import pallas as pl
from jax.experimental.pallas import tpu as pltpu
from jax.experimental.pallas import tpu_sc as plsc

_T = 64
_W = 3 * 224 * 224
_NC = 2
_NS = 16
_NW = _NC * _NS
_RPW = _T // _NW
_NCH = 4
_CH = _W // _NCH
_TOTAL = _RPW * _NCH


def _sc_body(vid, out, buf0, buf1, sem0, sem1):
    wid = lax.axis_index("s") * _NC + lax.axis_index("c")
    base_row = wid * _RPW
    bufs = (buf0, buf1)
    sems = (sem0, sem1)

    def in_off(q):
        r = base_row + q // _NCH
        return r * _W + (q % _NCH) * _CH

    def out_off(q, dup):
        r = base_row + q // _NCH
        return (2 * r + dup) * _W + (q % _NCH) * _CH

    handles = [None, None]
    handles[0] = pltpu.async_copy(vid.at[pl.ds(in_off(0), _CH)], bufs[0],
                                  sems[0])
    for q in range(_TOTAL):
        nxt = q + 1
        if nxt < _TOTAL:
            handles[nxt % 2] = pltpu.async_copy(
                vid.at[pl.ds(in_off(nxt), _CH)], bufs[nxt % 2], sems[nxt % 2])
        handles[q % 2].wait()
        pltpu.sync_copy(bufs[q % 2], out.at[pl.ds(out_off(q, 0), _CH)])
        pltpu.sync_copy(bufs[q % 2], out.at[pl.ds(out_off(q, 1), _CH)])


_sc_copy = functools.partial(
    pl.kernel,
    out_type=jax.ShapeDtypeStruct((2 * _T * _W,), jnp.float32),
    mesh=plsc.VectorSubcoreMesh(core_axis_name="c", subcore_axis_name="s"),
    scratch_types=[
        pltpu.VMEM((_CH,), jnp.float32),
        pltpu.VMEM((_CH,), jnp.float32),
        pltpu.SemaphoreType.DMA,
        pltpu.SemaphoreType.DMA,
    ],
)(_sc_body)


def kernel(video):
    vid = video.reshape(_T * _W)
    out = _sc_copy(vid)
    return out.reshape(2 * _T, 3, 224, 224)
